# baseline (speedup 1.0000x reference)
import functools
import jax
import jax.numpy as jnp
from jax import lax
from jax.experimental import pallas as pl
from jax.experimental.pallas import tpu as pltpu
from jax.experimental.pallas import tpu_sc as plsc

_B = 8192
_E = 16
_D = 2048
_NC = 2
_NS = 16
_NW = _NC * _NS
_BPW = _B // _NW
_G = _BPW // 16
_U = 8


def _sqrt16(x):
    i = plsc.bitcast(x, jnp.int32)
    i = jnp.int32(0x5F3759DF) - (i >> 1)
    y = plsc.bitcast(i, jnp.float32)
    for _ in range(3):
        y = y * (1.5 - 0.5 * x * y * y)
    return jnp.where(x > 0.0, x * y, 0.0)


def _sc_body(i0_hbm, i1_hbm, w0_hbm, w1_hbm, vt_hbm,
             out_hbm, part_hbm,
             table_v, i0_v, i1_v, w0_v, w1_v, outbuf_v, part_v):
    wid = lax.axis_index("s") * _NC + lax.axis_index("c")
    base = wid * _BPW
    lane = lax.iota(jnp.int32, 16)
    lane_row = lane * _D

    pltpu.sync_copy(vt_hbm, table_v)
    pltpu.sync_copy(i0_hbm.at[pl.ds(base, _BPW)], i0_v)
    pltpu.sync_copy(i1_hbm.at[pl.ds(base, _BPW)], i1_v)
    pltpu.sync_copy(w0_hbm.at[pl.ds(base, _BPW)], w0_v)
    pltpu.sync_copy(w1_hbm.at[pl.ds(base, _BPW)], w1_v)

    eff = jnp.zeros((16,), jnp.float32)
    for g in range(_G):
        idx0 = i0_v[pl.ds(g * 16, 16)]
        idx1 = i1_v[pl.ds(g * 16, 16)]
        w0 = w0_v[pl.ds(g * 16, 16)]
        w1 = w1_v[pl.ds(g * 16, 16)]
        total = w0 + w1
        denom = jnp.where(total > 0.0, total, 1.0)
        wn0 = w0 / denom
        wn1 = w1 / denom
        row0 = idx0 * _D
        row1 = idx1 * _D

        def chunk(c, acc):
            d0 = c * _U
            for j in range(_U):
                dj = d0 + j
                g0 = plsc.load_gather(table_v, [row0 + dj])
                g1 = plsc.load_gather(table_v, [row1 + dj])
                o = g0 * wn0 + g1 * wn1
                plsc.store_scatter(outbuf_v, [lane_row + dj], o)
                acc = acc + o * o
            return acc

        acc = lax.fori_loop(0, _D // _U, chunk, jnp.zeros((16,), jnp.float32))
        eff = eff + _sqrt16(acc)
        pltpu.sync_copy(outbuf_v,
                        out_hbm.at[pl.ds((base + g * 16) * _D, 16 * _D)])

    part_v[...] = eff
    pltpu.sync_copy(part_v, part_hbm.at[wid])


def kernel(expert_indices, expert_weights, vertices):
    i0 = expert_indices[:, 0]
    i1 = expert_indices[:, 1]
    w0 = expert_weights[:, 0]
    w1 = expert_weights[:, 1]

    f = pl.kernel(
        _sc_body,
        out_type=[
            jax.ShapeDtypeStruct((_B * _D,), jnp.float32),
            jax.ShapeDtypeStruct((_NW, 16), jnp.float32),
        ],
        mesh=plsc.VectorSubcoreMesh(core_axis_name="c", subcore_axis_name="s"),
        compiler_params=pltpu.CompilerParams(needs_layout_passes=False),
        scratch_types=[
            pltpu.VMEM((_E * _D,), jnp.float32),
            pltpu.VMEM((_BPW,), jnp.int32),
            pltpu.VMEM((_BPW,), jnp.int32),
            pltpu.VMEM((_BPW,), jnp.float32),
            pltpu.VMEM((_BPW,), jnp.float32),
            pltpu.VMEM((16 * _D,), jnp.float32),
            pltpu.VMEM((16,), jnp.float32),
        ],
    )
    path, parts = f(i0, i1, w0, w1, vertices.reshape(-1))
    eff = jnp.sum(parts) * (1.0 / _B)
    return path.reshape(_B, _D), eff


# speedup vs baseline: 1.5829x; 1.5829x over previous
import functools
import jax
import jax.numpy as jnp
from jax import lax
from jax.experimental import pallas as pl
from jax.experimental.pallas import tpu as pltpu
from jax.experimental.pallas import tpu_sc as plsc

_B = 8192
_E = 16
_D = 2048
_NC = 2
_NS = 16
_NW = _NC * _NS
_BPW = _B // _NW
_G = _BPW // 16
_U = 2
_NACC = 4


def _sqrt16(x):
    i = plsc.bitcast(x, jnp.int32)
    i = jnp.int32(0x5F3759DF) - (i >> 1)
    y = plsc.bitcast(i, jnp.float32)
    for _ in range(3):
        y = y * (1.5 - 0.5 * x * y * y)
    return jnp.where(x > 0.0, x * y, 0.0)


def _sc_body(i0_hbm, i1_hbm, w0_hbm, w1_hbm, vt_hbm,
             out_hbm, part_hbm,
             table_v, i0_v, i1_v, w0_v, w1_v, outbuf_a, outbuf_b, part_v,
             sem_a, sem_b):
    wid = lax.axis_index("s") * _NC + lax.axis_index("c")
    base = wid * _BPW
    lane = lax.iota(jnp.int32, 16)
    lane_row = lane * _D

    pltpu.sync_copy(vt_hbm, table_v)
    pltpu.sync_copy(i0_hbm.at[pl.ds(base, _BPW)], i0_v)
    pltpu.sync_copy(i1_hbm.at[pl.ds(base, _BPW)], i1_v)
    pltpu.sync_copy(w0_hbm.at[pl.ds(base, _BPW)], w0_v)
    pltpu.sync_copy(w1_hbm.at[pl.ds(base, _BPW)], w1_v)

    bufs = (outbuf_a, outbuf_b)
    sems = (sem_a, sem_b)
    pending = [None, None]
    eff = jnp.zeros((16,), jnp.float32)
    for g in range(_G):
        slot = g % 2
        outbuf_v = bufs[slot]
        if pending[slot] is not None:
            pending[slot].wait()
        idx0 = i0_v[pl.ds(g * 16, 16)]
        idx1 = i1_v[pl.ds(g * 16, 16)]
        w0 = w0_v[pl.ds(g * 16, 16)]
        w1 = w1_v[pl.ds(g * 16, 16)]
        total = w0 + w1
        denom = jnp.where(total > 0.0, total, 1.0)
        wn0 = w0 / denom
        wn1 = w1 / denom
        row0 = idx0 * _D
        row1 = idx1 * _D

        zero = jnp.zeros((16,), jnp.float32)

        @plsc.parallel_loop(0, _D, _NACC, unroll=_U, carry=(zero,) * _NACC)
        def accs(d0, accs):
            new = []
            for j in range(_NACC):
                dj = d0 + j
                g0 = plsc.load_gather(table_v, [row0 + dj])
                g1 = plsc.load_gather(table_v, [row1 + dj])
                o = g0 * wn0 + g1 * wn1
                plsc.store_scatter(outbuf_v, [lane_row + dj], o)
                new.append(accs[j] + o * o)
            return tuple(new)

        acc = accs[0] + accs[1] + accs[2] + accs[3]
        eff = eff + _sqrt16(acc)
        pending[slot] = pltpu.async_copy(
            outbuf_v, out_hbm.at[pl.ds((base + g * 16) * _D, 16 * _D)],
            sems[slot])

    for p in pending:
        if p is not None:
            p.wait()
    part_v[...] = eff
    pltpu.sync_copy(part_v, part_hbm.at[wid])


def kernel(expert_indices, expert_weights, vertices):
    i0 = expert_indices[:, 0]
    i1 = expert_indices[:, 1]
    w0 = expert_weights[:, 0]
    w1 = expert_weights[:, 1]

    f = pl.kernel(
        _sc_body,
        out_type=[
            jax.ShapeDtypeStruct((_B * _D,), jnp.float32),
            jax.ShapeDtypeStruct((_NW, 16), jnp.float32),
        ],
        mesh=plsc.VectorSubcoreMesh(core_axis_name="c", subcore_axis_name="s"),
        compiler_params=pltpu.CompilerParams(needs_layout_passes=False),
        scratch_types=[
            pltpu.VMEM((_E * _D,), jnp.float32),
            pltpu.VMEM((_BPW,), jnp.int32),
            pltpu.VMEM((_BPW,), jnp.int32),
            pltpu.VMEM((_BPW,), jnp.float32),
            pltpu.VMEM((_BPW,), jnp.float32),
            pltpu.VMEM((16 * _D,), jnp.float32),
            pltpu.VMEM((16 * _D,), jnp.float32),
            pltpu.VMEM((16,), jnp.float32),
            pltpu.SemaphoreType.DMA,
            pltpu.SemaphoreType.DMA,
        ],
    )
    path, parts = f(i0, i1, w0, w1, vertices.reshape(-1))
    eff = jnp.sum(parts) * (1.0 / _B)
    return path.reshape(_B, _D), eff


# speedup vs baseline: 7.0566x; 4.4579x over previous
import jax
import jax.numpy as jnp
from jax import lax
from jax.experimental import pallas as pl
from jax.experimental.pallas import tpu as pltpu
from jax.experimental.pallas import tpu_sc as plsc

_B = 8192
_E = 16
_D = 2048
_NC = 2
_NS = 16
_NW = _NC * _NS
_BPW = _B // _NW
_GRP = _BPW // 16
_U = 4
_TE = 2048


def _sc_body(i0_hbm, i1_hbm, w0_hbm, w1_hbm, vt_hbm, out_hbm,
             table_v, i0_v, i1_v, w0_v, w1_v, wn0_v, wn1_v,
             outbuf_a, outbuf_b, sem_a, sem_b):
    wid = lax.axis_index("s") * _NC + lax.axis_index("c")
    base = wid * _BPW

    pltpu.sync_copy(vt_hbm, table_v)
    pltpu.sync_copy(i0_hbm.at[pl.ds(base, _BPW)], i0_v.at[pl.ds(0, _BPW)])
    pltpu.sync_copy(i1_hbm.at[pl.ds(base, _BPW)], i1_v.at[pl.ds(0, _BPW)])
    pltpu.sync_copy(w0_hbm.at[pl.ds(base, _BPW)], w0_v.at[pl.ds(0, _BPW)])
    pltpu.sync_copy(w1_hbm.at[pl.ds(base, _BPW)], w1_v.at[pl.ds(0, _BPW)])

    for gg in range(_GRP):
        w0 = w0_v[pl.ds(gg * 16, 16)]
        w1 = w1_v[pl.ds(gg * 16, 16)]
        total = w0 + w1
        denom = jnp.where(total > 0.0, total, 1.0)
        wn0_v[pl.ds(gg * 16, 16)] = w0 / denom
        wn1_v[pl.ds(gg * 16, 16)] = w1 / denom

    bufs = (outbuf_a, outbuf_b)
    sems = (sem_a, sem_b)
    pending = [None, None]
    for g in range(_GRP):
        slot = g % 2
        outbuf_v = bufs[slot]
        if pending[slot] is not None:
            pending[slot].wait()

        def token_body(tl, carry, outbuf_v=outbuf_v, g=g):
            t = g * 16 + tl
            wn0s = wn0_v[pl.ds(t, 16)][0]
            wn1s = wn1_v[pl.ds(t, 16)][0]
            b0 = i0_v[pl.ds(t, 16)][0] * _D
            b1 = i1_v[pl.ds(t, 16)][0] * _D
            obase = tl * _D

            @plsc.parallel_loop(0, _D, 16, unroll=_U)
            def _(s):
                a = table_v[pl.ds(b0 + s, 16)]
                b = table_v[pl.ds(b1 + s, 16)]
                outbuf_v[pl.ds(obase + s, 16)] = a * wn0s + b * wn1s

            return carry

        lax.fori_loop(0, 16, token_body, 0)
        pending[slot] = pltpu.async_copy(
            outbuf_v, out_hbm.at[pl.ds((base + g * 16) * _D, 16 * _D)],
            sems[slot])

    for p in pending:
        if p is not None:
            p.wait()


def _eff_body(idx_ref, w_ref, v_ref, eff_ref):
    i = pl.program_id(0)
    v = v_ref[...]
    gram = lax.dot_general(v, v, (((1,), (1,)), ((), ())),
                           preferred_element_type=jnp.float32)
    idx = idx_ref[...]
    w = w_ref[...]
    total = w[:, 0:1] + w[:, 1:2]
    denom = jnp.where(total > 0.0, total, 1.0)
    wn = w / denom
    e = lax.broadcasted_iota(jnp.int32, (idx.shape[0], _E), 1)
    oh0 = jnp.where(idx[:, 0:1] == e, 1.0, 0.0)
    oh1 = jnp.where(idx[:, 1:2] == e, 1.0, 0.0)
    r0 = jnp.dot(oh0, gram, preferred_element_type=jnp.float32)
    g00 = jnp.sum(r0 * oh0, axis=1)
    g01 = jnp.sum(r0 * oh1, axis=1)
    r1 = jnp.dot(oh1, gram, preferred_element_type=jnp.float32)
    g11 = jnp.sum(r1 * oh1, axis=1)
    wn0 = wn[:, 0]
    wn1 = wn[:, 1]
    nsq = wn0 * wn0 * g00 + 2.0 * wn0 * wn1 * g01 + wn1 * wn1 * g11
    s = jnp.reshape(jnp.sum(jnp.sqrt(jnp.maximum(nsq, 0.0))), (1, 1))

    @pl.when(i == 0)
    def _():
        eff_ref[...] = s

    @pl.when(i > 0)
    def _():
        eff_ref[...] += s


def kernel(expert_indices, expert_weights, vertices):
    i0 = expert_indices[:, 0]
    i1 = expert_indices[:, 1]
    w0 = expert_weights[:, 0]
    w1 = expert_weights[:, 1]

    sc_f = pl.kernel(
        _sc_body,
        out_type=jax.ShapeDtypeStruct((_B * _D,), jnp.float32),
        mesh=plsc.VectorSubcoreMesh(core_axis_name="c", subcore_axis_name="s"),
        compiler_params=pltpu.CompilerParams(needs_layout_passes=False),
        scratch_types=[
            pltpu.VMEM((_E * _D,), jnp.float32),
            pltpu.VMEM((_BPW + 16,), jnp.int32),
            pltpu.VMEM((_BPW + 16,), jnp.int32),
            pltpu.VMEM((_BPW + 16,), jnp.float32),
            pltpu.VMEM((_BPW + 16,), jnp.float32),
            pltpu.VMEM((_BPW + 16,), jnp.float32),
            pltpu.VMEM((_BPW + 16,), jnp.float32),
            pltpu.VMEM((16 * _D,), jnp.float32),
            pltpu.VMEM((16 * _D,), jnp.float32),
            pltpu.SemaphoreType.DMA,
            pltpu.SemaphoreType.DMA,
        ],
    )
    path = sc_f(i0, i1, w0, w1, vertices.reshape(-1))

    effsum = pl.pallas_call(
        _eff_body,
        grid=(_B // _TE,),
        in_specs=[
            pl.BlockSpec((_TE, 2), lambda i: (i, 0)),
            pl.BlockSpec((_TE, 2), lambda i: (i, 0)),
            pl.BlockSpec((_E, _D), lambda i: (0, 0)),
        ],
        out_specs=pl.BlockSpec((1, 1), lambda i: (0, 0)),
        out_shape=jax.ShapeDtypeStruct((1, 1), jnp.float32),
    )(expert_indices, expert_weights, vertices)

    return path.reshape(_B, _D), effsum[0, 0] * (1.0 / _B)
